# Initial kernel scaffold; baseline (speedup 1.0000x reference)
import jax
import jax.numpy as jnp
from jax.experimental import pallas as pl


def kernel(feature1, feature2, h0, h1, s0, s1):
    raise NotImplementedError("write your pallas kernel here")


# speedup vs baseline: 3.5735x; 3.5735x over previous
import functools
import numpy as np
import jax
import jax.numpy as jnp
from jax import lax
from jax.experimental import pallas as pl
from jax.experimental.pallas import tpu as pltpu

DD = 8192
N1 = 64
N2 = 128
RBLK = 128
DTILE = 512


def _fft_consts():
    n2 = np.arange(N2)
    k2 = np.arange(N2)
    n1 = np.arange(N1)
    k1 = np.arange(N1)
    W2 = np.exp(-2j * np.pi * np.outer(n2, k2) / N2)
    T = np.exp(-2j * np.pi * np.outer(n1, k2) / DD)
    W1 = np.exp(-2j * np.pi * np.outer(n1, k1) / N1)
    W1i = np.exp(2j * np.pi * np.outer(k1, n1) / N1)
    T2 = np.exp(2j * np.pi * np.outer(k2, n1) / DD)
    W2i = np.exp(2j * np.pi * np.outer(k2, n2) / N2) / (2.0 * DD)
    f32 = lambda a: np.asarray(a, np.float32)
    return tuple(
        f32(x)
        for x in (
            W2.real, W2.imag, T.real, T.imag, W1.real, W1.imag,
            W1i.real, W1i.imag, T2.real, T2.imag, W2i.real, W2i.imag,
            np.eye(N1),
        )
    )


def _dg(x, w, axis):
    return lax.dot_general(
        x, w, (((axis,), (0,)), ((), ())), preferred_element_type=jnp.float32
    )


def _mcb_body(
    f1_ref, f2_ref, h0_ref, s0_ref, h1_ref, s1_ref,
    w2r, w2i, tr, ti, w1r, w1i, w1ir, w1ii, t2r, t2i, w2ir, w2ii, i64,
    out_ref, sk1_ref, sk2_ref,
):
    nf1 = f1_ref.shape[1]
    nf2 = f2_ref.shape[1]

    def sketch_tile(f, h, s, dt):
        base = dt * DTILE
        cols = lax.broadcasted_iota(jnp.int32, (h.shape[0], DTILE), 1) + base
        p = jnp.where(cols == h, s, jnp.float32(0.0))
        return _dg(f, p, 1)

    f1 = f1_ref[...]
    f2 = f2_ref[...]
    h0 = h0_ref[...]
    s0 = s0_ref[...]
    h1 = h1_ref[...]
    s1 = s1_ref[...]

    def body(dt, _):
        sk1_ref[:, pl.ds(dt * DTILE, DTILE)] = sketch_tile(f1, h0, s0, dt)
        sk2_ref[:, pl.ds(dt * DTILE, DTILE)] = sketch_tile(f2, h1, s1, dt)
        return 0

    lax.fori_loop(0, DD // DTILE, body, 0)

    r = out_ref.shape[0]
    x3r = sk1_ref[...].reshape(r, N2, N1)
    x3i = sk2_ref[...].reshape(r, N2, N1)

    br = _dg(x3r, w2r[...], 1) - _dg(x3i, w2i[...], 1)
    bi = _dg(x3r, w2i[...], 1) + _dg(x3i, w2r[...], 1)
    cr = br * tr[...][None] - bi * ti[...][None]
    ci = br * ti[...][None] + bi * tr[...][None]
    zr = _dg(cr, w1r[...], 1) - _dg(ci, w1i[...], 1)
    zi = _dg(cr, w1i[...], 1) + _dg(ci, w1r[...], 1)
    qr = zr * zr - zi * zi
    qi = 2.0 * zr * zi
    fr = _dg(qr, w1ir[...], 2) - _dg(qi, w1ii[...], 2)
    fi = _dg(qr, w1ii[...], 2) + _dg(qi, w1ir[...], 2)
    gr = fr * t2r[...][None] - fi * t2i[...][None]
    gi = fr * t2i[...][None] + fi * t2r[...][None]
    hi = _dg(gr, w2ii[...], 1) + _dg(gi, w2ir[...], 1)
    out3 = _dg(hi, i64[...], 1)
    out_ref[...] = out3.reshape(r, DD)


@jax.jit
def kernel(feature1, feature2, h0, h1, s0, s1):
    b = feature1.shape[0]
    nf1 = feature1.shape[1]
    nf2 = feature2.shape[1]
    consts = _fft_consts()
    h0c = h0.astype(jnp.int32).reshape(nf1, 1)
    h1c = h1.astype(jnp.int32).reshape(nf2, 1)
    s0c = s0.astype(jnp.float32).reshape(nf1, 1)
    s1c = s1.astype(jnp.float32).reshape(nf2, 1)

    grid = (b // RBLK,)
    full = lambda shape: pl.BlockSpec(shape, lambda i: (0,) * len(shape))
    in_specs = [
        pl.BlockSpec((RBLK, nf1), lambda i: (i, 0)),
        pl.BlockSpec((RBLK, nf2), lambda i: (i, 0)),
        full((nf1, 1)),
        full((nf1, 1)),
        full((nf2, 1)),
        full((nf2, 1)),
    ] + [full(c.shape) for c in consts]

    out = pl.pallas_call(
        _mcb_body,
        grid=grid,
        in_specs=in_specs,
        out_specs=pl.BlockSpec((RBLK, DD), lambda i: (i, 0)),
        out_shape=jax.ShapeDtypeStruct((b, DD), jnp.float32),
        scratch_shapes=[
            pltpu.VMEM((RBLK, DD), jnp.float32),
            pltpu.VMEM((RBLK, DD), jnp.float32),
        ],
        compiler_params=pltpu.CompilerParams(
            dimension_semantics=("arbitrary",),
        ),
    )(feature1, feature2, h0c, s0c, h1c, s1c, *consts)
    return out
